# baseline (speedup 1.0000x reference)
import functools
import math

import jax
import jax.numpy as jnp
import numpy as np
from jax import lax
from jax.experimental import pallas as pl
from jax.experimental.pallas import tpu as pltpu
from jax.experimental.pallas import tpu_sc as plsc

_B, _H, _S, _D = 4, 16, 2048, 128
_NROWS = _B * _H * _S

_CENTROIDS = np.array(
    [-2.7326, -2.069, -1.618, -1.2562, -0.9423, -0.6568, -0.3881, -0.1284,
     0.1284, 0.3881, 0.6568, 0.9423, 1.2562, 1.618, 2.069, 2.7326],
    dtype=np.float32)
_BOUNDS = ((_CENTROIDS[:-1] + _CENTROIDS[1:]) / 2).astype(np.float32)
_PB = _BOUNDS[8:]
_C8 = float(_CENTROIDS[8])
_DCP = [float(x) for x in (_CENTROIDS[9:] - _CENTROIDS[8:15])]
_PB2D = [float(x) for x in (_PB.astype(np.float64) ** 2 / _D)]
_INV_SQRT_D = float(np.float32(1.0 / math.sqrt(_D)))

_NW = 32
_CHUNK = 128
_CS = _CHUNK * _D


def _rsqrt_newton(ssc):
    ii = plsc.bitcast(ssc, jnp.int32)
    ii = 0x5F3759DF - lax.shift_right_logical(ii, 1)
    y = plsc.bitcast(ii, jnp.float32)
    for _ in range(3):
        y = y * (1.5 - 0.5 * ssc * y * y)
    return y


def _sc_compute_chunk(inb_b, outb_b, rows_off):
    def group_body(g, carry):
        idx0 = rows_off + g * (16 * _D)
        def p1(c4, sums):
            sm, ssm = sums
            for cc in range(32):
                x = plsc.load_gather(inb_b, [idx0 + (c4 * 32 + cc)])
                sm = sm + x
                ssm = ssm + x * x
            return sm, ssm

        zero = jnp.zeros((16,), jnp.float32)
        sm, ssm = lax.fori_loop(0, 4, p1, (zero, zero))
        mean = sm * (1.0 / _D)
        ssc = jnp.maximum(ssm - mean * sm, 1e-30)
        mag = jnp.maximum(ssc * _rsqrt_newton(ssc), 1e-8)
        rm = mag * _INV_SQRT_D
        tbs = [pb2d * ssc for pb2d in _PB2D]

        def p2(c4, carry2):
            for cc in range(32):
                idx = idx0 + (c4 * 32 + cc)
                x = plsc.load_gather(inb_b, [idx])
                xc = x - mean
                t = xc * xc
                acc = jnp.full((16,), _C8, jnp.float32)
                for j in range(7):
                    acc = acc + jnp.where(t > tbs[j], _DCP[j], 0.0)
                val = jnp.where(xc > 0, acc, -acc) * rm + mean
                plsc.store_scatter(outb_b, [idx], val)
            return carry2

        lax.fori_loop(0, 4, p2, 0)
        return carry

    lax.fori_loop(0, _CHUNK // 16, group_body, 0)


def _sc_make(n_rows):
    rpw = n_rows // _NW
    nch = rpw // _CHUNK
    assert rpw % _CHUNK == 0 and nch % 2 == 0
    mesh = plsc.VectorSubcoreMesh(core_axis_name="c", subcore_axis_name="s")
    out = jax.ShapeDtypeStruct((n_rows * _D,), jnp.float32)

    @functools.partial(
        pl.kernel, mesh=mesh,
        out_type=[out, out],
        compiler_params=pltpu.CompilerParams(needs_layout_passes=False),
        scratch_types=[
            pltpu.VMEM((_CS,), jnp.float32),
            pltpu.VMEM((_CS,), jnp.float32),
            pltpu.VMEM((_CS,), jnp.float32),
            pltpu.VMEM((_CS,), jnp.float32),
            pltpu.SemaphoreType.DMA,
            pltpu.SemaphoreType.DMA,
            pltpu.SemaphoreType.DMA,
            pltpu.SemaphoreType.DMA,
        ])
    def sc_kernel(k_hbm, v_hbm, ko_hbm, vo_hbm, inb0, inb1, outb0, outb1,
                  is0, is1, os0, os1):
        cid = lax.axis_index("c")
        sid = lax.axis_index("s")
        wid = sid * 2 + cid
        base = wid * (rpw * _D)
        rows_off = lax.iota(jnp.int32, 16) * _D
        inbs = (inb0, inb1)
        outbs = (outb0, outb1)
        isems = (is0, is1)
        osems = (os0, os1)

        def phase(src, dst):
            pltpu.async_copy(src.at[pl.ds(base, _CS)], inbs[0], isems[0])

            def pair_body(p, carry):
                for b in (0, 1):
                    i = 2 * p + b
                    nb = 1 - b
                    nxt = jnp.minimum(i + 1, nch - 1)
                    pltpu.async_copy(
                        src.at[pl.ds(base + nxt * _CS, _CS)],
                        inbs[nb], isems[nb])
                    pltpu.make_async_copy(
                        src.at[pl.ds(base + i * _CS, _CS)],
                        inbs[b], isems[b]).wait()
                    @pl.when(i >= 2)
                    def _():
                        pltpu.make_async_copy(
                            outbs[b],
                            dst.at[pl.ds(base + (i - 2) * _CS, _CS)],
                            osems[b]).wait()
                    _sc_compute_chunk(inbs[b], outbs[b], rows_off)
                    pltpu.async_copy(
                        outbs[b],
                        dst.at[pl.ds(base + i * _CS, _CS)], osems[b])
                return carry

            lax.fori_loop(0, nch // 2, pair_body, 0)
            pltpu.make_async_copy(
                src.at[pl.ds(base, _CS)], inbs[0], isems[0]).wait()
            pltpu.make_async_copy(
                outbs[0],
                dst.at[pl.ds(base + (nch - 2) * _CS, _CS)], osems[0]).wait()
            pltpu.make_async_copy(
                outbs[1],
                dst.at[pl.ds(base + (nch - 1) * _CS, _CS)], osems[1]).wait()

        phase(k_hbm, ko_hbm)
        phase(v_hbm, vo_hbm)

    return sc_kernel


@jax.jit
def _run_sc(k1d, v1d):
    return _sc_make(_NROWS)(k1d, v1d)


def kernel(input_pos, k_val, v_val, k_packed, v_packed, k_mag, v_mag,
           k_mean, v_mean):
    shape = k_val.shape
    ko, vo = _run_sc(k_val.reshape(-1), v_val.reshape(-1))
    return ko.reshape(shape), vo.reshape(shape)


# speedup vs baseline: 6.4942x; 6.4942x over previous
import functools
import math

import jax
import jax.numpy as jnp
import numpy as np
from jax import lax
from jax.experimental import pallas as pl
from jax.experimental.pallas import tpu as pltpu
from jax.experimental.pallas import tpu_sc as plsc

_B, _H, _S, _D = 4, 16, 2048, 128
_NROWS = _B * _H * _S

_CENTROIDS = np.array(
    [-2.7326, -2.069, -1.618, -1.2562, -0.9423, -0.6568, -0.3881, -0.1284,
     0.1284, 0.3881, 0.6568, 0.9423, 1.2562, 1.618, 2.069, 2.7326],
    dtype=np.float32)
_BOUNDS = ((_CENTROIDS[:-1] + _CENTROIDS[1:]) / 2).astype(np.float32)
_PB = _BOUNDS[8:]
_C8 = float(_CENTROIDS[8])
_DCP = [float(x) for x in (_CENTROIDS[9:] - _CENTROIDS[8:15])]
_PB2D = [float(x) for x in (_PB.astype(np.float64) ** 2 / _D)]
_INV_SQRT_D = float(np.float32(1.0 / math.sqrt(_D)))

_NW = 32
_CHUNK = 128
_CS = _CHUNK * _D


def _rsqrt_newton(ssc):
    ii = lax.bitcast_convert_type(ssc, jnp.int32)
    ii = 0x5F3759DF - lax.shift_right_logical(ii, 1)
    y = lax.bitcast_convert_type(ii, jnp.float32)
    for _ in range(3):
        y = y * (1.5 - 0.5 * ssc * y * y)
    return y


def _sc_compute_chunk(inb_b, outb_b, rows_off):
    del rows_off

    def row_body(r, carry):
        base = r * _D
        xs = [inb_b[pl.ds(base + 16 * i, 16)] for i in range(8)]
        sm = ((xs[0] + xs[1]) + (xs[2] + xs[3])) + (
            (xs[4] + xs[5]) + (xs[6] + xs[7]))
        sq = [x * x for x in xs]
        sqm = ((sq[0] + sq[1]) + (sq[2] + sq[3])) + (
            (sq[4] + sq[5]) + (sq[6] + sq[7]))
        tot = jnp.sum(sm)
        tot2 = jnp.sum(sqm)
        mean = tot * (1.0 / _D)
        ssc = jnp.maximum(tot2 - mean * tot, 1e-30)
        mag = jnp.maximum(ssc * _rsqrt_newton(ssc), 1e-8)
        rm = mag * _INV_SQRT_D
        mean_v = jnp.full((16,), 1.0, jnp.float32) * mean
        rm_v = jnp.full((16,), 1.0, jnp.float32) * rm
        tbs = [jnp.full((16,), 1.0, jnp.float32) * (pb2d * ssc)
               for pb2d in _PB2D]
        for i in range(8):
            xc = xs[i] - mean_v
            t = xc * xc
            acc = jnp.full((16,), _C8, jnp.float32)
            for j in range(7):
                acc = acc + jnp.where(t > tbs[j], _DCP[j], 0.0)
            val = jnp.where(xc > 0, acc, -acc) * rm_v + mean_v
            outb_b[pl.ds(base + 16 * i, 16)] = val
        return carry

    lax.fori_loop(0, _CHUNK, row_body, 0)


def _sc_make(n_rows):
    rpw = n_rows // _NW
    nch = rpw // _CHUNK
    assert rpw % _CHUNK == 0 and nch % 2 == 0
    mesh = plsc.VectorSubcoreMesh(core_axis_name="c", subcore_axis_name="s")
    out = jax.ShapeDtypeStruct((n_rows * _D,), jnp.float32)

    @functools.partial(
        pl.kernel, mesh=mesh,
        out_type=[out, out],
        compiler_params=pltpu.CompilerParams(needs_layout_passes=False),
        scratch_types=[
            pltpu.VMEM((_CS,), jnp.float32),
            pltpu.VMEM((_CS,), jnp.float32),
            pltpu.VMEM((_CS,), jnp.float32),
            pltpu.VMEM((_CS,), jnp.float32),
            pltpu.SemaphoreType.DMA,
            pltpu.SemaphoreType.DMA,
            pltpu.SemaphoreType.DMA,
            pltpu.SemaphoreType.DMA,
        ])
    def sc_kernel(k_hbm, v_hbm, ko_hbm, vo_hbm, inb0, inb1, outb0, outb1,
                  is0, is1, os0, os1):
        cid = lax.axis_index("c")
        sid = lax.axis_index("s")
        wid = sid * 2 + cid
        base = wid * (rpw * _D)
        rows_off = lax.iota(jnp.int32, 16) * _D
        inbs = (inb0, inb1)
        outbs = (outb0, outb1)
        isems = (is0, is1)
        osems = (os0, os1)

        def phase(src, dst):
            pltpu.async_copy(src.at[pl.ds(base, _CS)], inbs[0], isems[0])

            def pair_body(p, carry):
                for b in (0, 1):
                    i = 2 * p + b
                    nb = 1 - b
                    nxt = jnp.minimum(i + 1, nch - 1)
                    pltpu.async_copy(
                        src.at[pl.ds(base + nxt * _CS, _CS)],
                        inbs[nb], isems[nb])
                    pltpu.make_async_copy(
                        src.at[pl.ds(base + i * _CS, _CS)],
                        inbs[b], isems[b]).wait()
                    @pl.when(i >= 2)
                    def _():
                        pltpu.make_async_copy(
                            outbs[b],
                            dst.at[pl.ds(base + (i - 2) * _CS, _CS)],
                            osems[b]).wait()
                    _sc_compute_chunk(inbs[b], outbs[b], rows_off)
                    pltpu.async_copy(
                        outbs[b],
                        dst.at[pl.ds(base + i * _CS, _CS)], osems[b])
                return carry

            lax.fori_loop(0, nch // 2, pair_body, 0)
            pltpu.make_async_copy(
                src.at[pl.ds(base, _CS)], inbs[0], isems[0]).wait()
            pltpu.make_async_copy(
                outbs[0],
                dst.at[pl.ds(base + (nch - 2) * _CS, _CS)], osems[0]).wait()
            pltpu.make_async_copy(
                outbs[1],
                dst.at[pl.ds(base + (nch - 1) * _CS, _CS)], osems[1]).wait()

        phase(k_hbm, ko_hbm)
        phase(v_hbm, vo_hbm)

    return sc_kernel


@jax.jit
def _run_sc(k1d, v1d):
    return _sc_make(_NROWS)(k1d, v1d)


def kernel(input_pos, k_val, v_val, k_packed, v_packed, k_mag, v_mag,
           k_mean, v_mean):
    shape = k_val.shape
    ko, vo = _run_sc(k_val.reshape(-1), v_val.reshape(-1))
    return ko.reshape(shape), vo.reshape(shape)


# speedup vs baseline: 12.3292x; 1.8985x over previous
import functools
import math

import jax
import jax.numpy as jnp
import numpy as np
from jax import lax
from jax.experimental import pallas as pl
from jax.experimental.pallas import tpu as pltpu
from jax.experimental.pallas import tpu_sc as plsc

_B, _H, _S, _D = 4, 16, 2048, 128
_NROWS = _B * _H * _S

_CENTROIDS = np.array(
    [-2.7326, -2.069, -1.618, -1.2562, -0.9423, -0.6568, -0.3881, -0.1284,
     0.1284, 0.3881, 0.6568, 0.9423, 1.2562, 1.618, 2.069, 2.7326],
    dtype=np.float32)
_BOUNDS = ((_CENTROIDS[:-1] + _CENTROIDS[1:]) / 2).astype(np.float32)
_PB = _BOUNDS[8:]
_C8 = float(_CENTROIDS[8])
_DCP = [float(x) for x in (_CENTROIDS[9:] - _CENTROIDS[8:15])]
_PB2D = [float(x) for x in (_PB.astype(np.float64) ** 2 / _D)]
_INV_SQRT_D = float(np.float32(1.0 / math.sqrt(_D)))

_NW = 32
_CHUNK = 128
_CS = _CHUNK * _D


def _rsqrt_newton(ssc):
    ii = lax.bitcast_convert_type(ssc, jnp.int32)
    ii = 0x5F3759DF - lax.shift_right_logical(ii, 1)
    y = lax.bitcast_convert_type(ii, jnp.float32)
    for _ in range(3):
        y = y * (1.5 - 0.5 * ssc * y * y)
    return y


def _sc_compute_chunk(inb_b, outb_b):
    def row_body(r, carry):
        base = r * _D
        xs = [inb_b[pl.ds(base + 16 * i, 16)] for i in range(8)]
        sm = ((xs[0] + xs[1]) + (xs[2] + xs[3])) + (
            (xs[4] + xs[5]) + (xs[6] + xs[7]))
        sq = [x * x for x in xs]
        sqm = ((sq[0] + sq[1]) + (sq[2] + sq[3])) + (
            (sq[4] + sq[5]) + (sq[6] + sq[7]))
        tot = jnp.sum(sm)
        tot2 = jnp.sum(sqm)
        mean = tot * (1.0 / _D)
        ssc = jnp.maximum(tot2 - mean * tot, 1e-30)
        mag = jnp.maximum(ssc * _rsqrt_newton(ssc), 1e-8)
        rm = mag * _INV_SQRT_D
        ones = jnp.full((16,), 1.0, jnp.float32)
        mean_v = ones * mean
        rm_v = ones * rm
        tbs = [ones * (pb2d * ssc) for pb2d in _PB2D]
        for i in range(8):
            xc = xs[i] - mean_v
            t = xc * xc
            acc = jnp.full((16,), _C8, jnp.float32)
            for j in range(7):
                acc = acc + jnp.where(t > tbs[j], _DCP[j], 0.0)
            val = jnp.where(xc > 0, acc, -acc) * rm_v + mean_v
            outb_b[pl.ds(base + 16 * i, 16)] = val
        return carry

    lax.fori_loop(0, _CHUNK, row_body, 0)


def _sc_make(n_rows):
    rpw = n_rows // _NW
    nch = rpw // _CHUNK
    assert rpw % _CHUNK == 0 and nch % 2 == 0
    mesh = plsc.VectorSubcoreMesh(core_axis_name="c", subcore_axis_name="s")
    out = jax.ShapeDtypeStruct((n_rows * _D,), jnp.float32)

    @functools.partial(
        pl.kernel, mesh=mesh,
        out_type=out,
        compiler_params=pltpu.CompilerParams(needs_layout_passes=False),
        scratch_types=[
            pltpu.VMEM((_CS,), jnp.float32),
            pltpu.VMEM((_CS,), jnp.float32),
            pltpu.VMEM((_CS,), jnp.float32),
            pltpu.VMEM((_CS,), jnp.float32),
            pltpu.SemaphoreType.DMA,
            pltpu.SemaphoreType.DMA,
            pltpu.SemaphoreType.DMA,
            pltpu.SemaphoreType.DMA,
        ])
    def sc_kernel(src, dst, inb0, inb1, outb0, outb1, is0, is1, os0, os1):
        cid = lax.axis_index("c")
        sid = lax.axis_index("s")
        wid = sid * 2 + cid
        base = wid * (rpw * _D)
        inbs = (inb0, inb1)
        outbs = (outb0, outb1)
        isems = (is0, is1)
        osems = (os0, os1)

        pltpu.async_copy(src.at[pl.ds(base, _CS)], inbs[0], isems[0])

        def pair_body(p, carry):
            for b in (0, 1):
                i = 2 * p + b
                nb = 1 - b
                nxt = jnp.minimum(i + 1, nch - 1)
                pltpu.async_copy(
                    src.at[pl.ds(base + nxt * _CS, _CS)],
                    inbs[nb], isems[nb])
                pltpu.make_async_copy(
                    src.at[pl.ds(base + i * _CS, _CS)],
                    inbs[b], isems[b]).wait()
                @pl.when(i >= 2)
                def _():
                    pltpu.make_async_copy(
                        outbs[b],
                        dst.at[pl.ds(base + (i - 2) * _CS, _CS)],
                        osems[b]).wait()
                _sc_compute_chunk(inbs[b], outbs[b])
                pltpu.async_copy(
                    outbs[b],
                    dst.at[pl.ds(base + i * _CS, _CS)], osems[b])
            return carry

        lax.fori_loop(0, nch // 2, pair_body, 0)
        pltpu.make_async_copy(
            src.at[pl.ds(base, _CS)], inbs[0], isems[0]).wait()
        pltpu.make_async_copy(
            outbs[0],
            dst.at[pl.ds(base + (nch - 2) * _CS, _CS)], osems[0]).wait()
        pltpu.make_async_copy(
            outbs[1],
            dst.at[pl.ds(base + (nch - 1) * _CS, _CS)], osems[1]).wait()

    return sc_kernel


def _quant_dequant(x):
    mean = jnp.mean(x, axis=-1, keepdims=True)
    xc = x - mean
    ss = jnp.sum(xc * xc, axis=-1, keepdims=True)
    mag = jnp.maximum(jnp.sqrt(ss), 1e-8)
    rm = mag * _INV_SQRT_D
    a = jnp.abs(xc)
    acc = jnp.broadcast_to(_C8 * rm, x.shape)
    for j in range(7):
        acc = acc + jnp.where(a > float(_PB[j]) * rm, _DCP[j] * rm, 0.0)
    return jnp.where(xc > 0, acc, -acc) + mean


def _tc_body(v_ref, vo_ref):
    vo_ref[...] = _quant_dequant(v_ref[...])


def _tc_run(v2d):
    n = v2d.shape[0]
    blk = 2048
    spec = pl.BlockSpec((blk, _D), lambda i: (i, 0))
    return pl.pallas_call(
        _tc_body,
        grid=(n // blk,),
        in_specs=[spec],
        out_specs=spec,
        out_shape=jax.ShapeDtypeStruct((n, _D), jnp.float32),
    )(v2d)


@jax.jit
def _run(k1d, v2d):
    ko = _sc_make(_NROWS)(k1d)
    vo = _tc_run(v2d)
    return ko, vo


def kernel(input_pos, k_val, v_val, k_packed, v_packed, k_mag, v_mag,
           k_mean, v_mean):
    shape = k_val.shape
    ko, vo = _run(k_val.reshape(-1), v_val.reshape(-1, _D))
    return ko.reshape(shape), vo.reshape(shape)
